# baseline (speedup 1.0000x reference)
import jax
import jax.numpy as jnp
from jax.experimental import pallas as pl
from jax.experimental.pallas import tpu as pltpu

M, K, N = 1024, 100000, 16
BM = 16
NBUF = 4
NBLK = M // BM
NROUNDS = NBLK // NBUF


def _matmul_kernel(x_hbm, wt_ref, o_ref, *scratch):
    bufs = scratch[:NBUF]
    sems = scratch[NBUF:]

    def copy_in(b, s):
        return pltpu.make_async_copy(
            x_hbm.at[pl.ds(b * BM, BM), :], bufs[s], sems[s])

    for s in range(NBUF):
        copy_in(s, s).start()

    def body(r, _):
        base = r * NBUF
        for s in range(NBUF):
            b = base + s
            copy_in(b, s).wait()
            o_ref[pl.ds(b * BM, BM), :] = jax.lax.dot_general(
                bufs[s][...], wt_ref[...],
                dimension_numbers=(((1,), (1,)), ((), ())),
                preferred_element_type=jnp.float32)

            @pl.when(b + NBUF < NBLK)
            def _prefetch():
                copy_in(b + NBUF, s).start()

        return 0

    jax.lax.fori_loop(0, NROUNDS, body, 0)


def kernel(x, W):
    wt = W.T
    return pl.pallas_call(
        _matmul_kernel,
        in_specs=[
            pl.BlockSpec(memory_space=pltpu.MemorySpace.HBM),
            pl.BlockSpec((N, K), lambda: (0, 0)),
        ],
        out_specs=pl.BlockSpec((M, N), lambda: (0, 0)),
        out_shape=jax.ShapeDtypeStruct((M, N), jnp.float32),
        scratch_shapes=(
            [pltpu.VMEM((BM, K), jnp.float32) for _ in range(NBUF)]
            + [pltpu.SemaphoreType.DMA for _ in range(NBUF)]
        ),
    )(x, wt)
